# Initial kernel scaffold; baseline (speedup 1.0000x reference)
import jax
import jax.numpy as jnp
from jax.experimental import pallas as pl


def kernel(x, edge_index, Wq, bq, Wk, bk, Wm, bm, Watt, Wmsg, prior, Waggr, baggr, skip_logit, gamma, beta):
    raise NotImplementedError("write your pallas kernel here")


# speedup vs baseline: 19.5772x; 19.5772x over previous
import functools

import jax
import jax.numpy as jnp
from jax import lax
from jax.experimental import pallas as pl
from jax.experimental.pallas import tpu as pltpu
from jax.experimental.pallas import tpu_sc as plsc

_N = 10000
_E = 320000
_D = 128
_H = 8
_C = 16

_NC = 2
_NS = 16
_NW = _NC * _NS
_EPW = _E // _NW
_CHUNK = 40
_NSTEP = _EPW // _CHUNK
_ROWS_PT = _N // _NS
_ACC_W = _D + 16

_ROW_BLK = 1000


def _proj_body(x_ref, wq_ref, bq_ref, wk_ref, bk_ref, wm_ref, bm_ref,
               bwa_ref, bwm_ref, q_ref, kt_ref, mt_ref):
    xb = x_ref[...]
    f32 = jnp.float32
    q_ref[...] = jnp.dot(xb, wq_ref[...], preferred_element_type=f32) + bq_ref[...]
    kb = jnp.dot(xb, wk_ref[...], preferred_element_type=f32) + bk_ref[...]
    kt_ref[...] = jnp.dot(kb, bwa_ref[...], preferred_element_type=f32)
    mb = jnp.dot(xb, wm_ref[...], preferred_element_type=f32) + bm_ref[...]
    mt_ref[...] = jnp.dot(mb, bwm_ref[...], preferred_element_type=f32)


def _edge_body(kt_hbm, q_hbm, mt_hbm, src_hbm, dst_hbm, zeros_hbm, out_hbm,
               src_v, dst_v, ktb, qb, mtb, ob, acc, sem0, sem1, sem2):
    cid = lax.axis_index("c")
    sid = lax.axis_index("s")
    wid = cid * _NS + sid

    pltpu.sync_copy(zeros_hbm.at[pl.ds(sid * _ROWS_PT, _ROWS_PT)],
                    acc.at[pl.ds(sid * _ROWS_PT, _ROWS_PT)])
    plsc.subcore_barrier()

    base = wid * _EPW
    lane = lax.iota(jnp.int32, 16)

    def step(g, carry):
        off = base + g * _CHUNK
        pltpu.sync_copy(src_hbm.at[pl.ds(off, _CHUNK)], src_v)
        pltpu.sync_copy(dst_hbm.at[pl.ds(off, _CHUNK)], dst_v)
        cp0 = pltpu.async_copy(kt_hbm.at[src_v], ktb, sem0)
        cp1 = pltpu.async_copy(q_hbm.at[dst_v], qb, sem1)
        cp2 = pltpu.async_copy(mt_hbm.at[src_v], mtb, sem2)
        cp0.wait()
        cp1.wait()
        cp2.wait()

        def edge(e, c2):
            dv = jnp.zeros((16,), jnp.float32)
            for h in range(_H):
                sl = pl.ds(h * 16, 16)
                ktv = ktb[e, sl]
                qv = qb[e, sl]
                s = jnp.sum(ktv * qv)
                ev = jnp.exp(jnp.full((16,), s, jnp.float32))
                ob[e, sl] = ev * mtb[e, sl]
                dv = jnp.where(lane == h, ev, dv)
            ob[e, pl.ds(_D, 16)] = dv
            return c2

        lax.fori_loop(0, _CHUNK, edge, 0)
        pltpu.sync_copy(ob, acc.at[dst_v], add=True)
        return carry

    lax.fori_loop(0, _NSTEP, step, 0)
    plsc.subcore_barrier()
    pltpu.sync_copy(acc.at[pl.ds(sid * _ROWS_PT, _ROWS_PT)],
                    out_hbm.at[cid, pl.ds(sid * _ROWS_PT, _ROWS_PT)])


@functools.cache
def _edge_kernel():
    return functools.partial(
        pl.kernel,
        mesh=plsc.VectorSubcoreMesh(core_axis_name="c", subcore_axis_name="s"),
        compiler_params=pltpu.CompilerParams(use_tc_tiling_on_sc=False,
                                             needs_layout_passes=False),
        out_type=jax.ShapeDtypeStruct((_NC, _N, _ACC_W), jnp.float32),
        scratch_types=[
            pltpu.VMEM((_CHUNK,), jnp.int32),
            pltpu.VMEM((_CHUNK,), jnp.int32),
            pltpu.VMEM((_CHUNK, _D), jnp.float32),
            pltpu.VMEM((_CHUNK, _D), jnp.float32),
            pltpu.VMEM((_CHUNK, _D), jnp.float32),
            pltpu.VMEM((_CHUNK, _ACC_W), jnp.float32),
            pltpu.VMEM_SHARED((_N, _ACC_W), jnp.float32),
            pltpu.SemaphoreType.DMA,
            pltpu.SemaphoreType.DMA,
            pltpu.SemaphoreType.DMA,
        ],
    )(_edge_body)


def _out_body(acc_ref, x_ref, waggr_ref, baggr_ref, slog_ref, gamma_ref,
              beta_ref, exp_ref, o_ref):
    a0 = acc_ref[0]
    a1 = acc_ref[1]
    num = a0[:, :_D] + a1[:, :_D]
    den = a0[:, _D:] + a1[:, _D:]
    den_exp = jnp.dot(den, exp_ref[...], preferred_element_type=jnp.float32)
    pooled = num / jnp.maximum(den_exp, 1e-30)
    act = jax.nn.gelu(pooled)
    aggr = jnp.dot(act, waggr_ref[...], preferred_element_type=jnp.float32)
    aggr = aggr + baggr_ref[...]
    gate = jax.nn.sigmoid(slog_ref[0, 0])
    out = gate * aggr + (1.0 - gate) * x_ref[...]
    mean = jnp.mean(out, axis=-1, keepdims=True)
    cent = out - mean
    var = jnp.mean(cent * cent, axis=-1, keepdims=True)
    out = cent * lax.rsqrt(var + 1e-3)
    o_ref[...] = gamma_ref[...] * out + beta_ref[...]


def kernel(x, edge_index, Wq, bq, Wk, bk, Wm, bm, Watt, Wmsg, prior, Waggr,
           baggr, skip_logit, gamma, beta):
    src = edge_index[0]
    dst = edge_index[1]

    eyeh = jnp.eye(_H, dtype=jnp.float32)
    watt_s = Watt * (prior / jnp.sqrt(jnp.float32(_C)))[:, None, None]
    bwa = jnp.einsum('hcd,hg->hcgd', watt_s, eyeh).reshape(_D, _D)
    bwm = jnp.einsum('hcd,hg->hcgd', Wmsg, eyeh).reshape(_D, _D)

    full = pl.BlockSpec((_D, _D), lambda i: (0, 0))
    row = pl.BlockSpec((1, _D), lambda i: (0, 0))
    blk = pl.BlockSpec((_ROW_BLK, _D), lambda i: (i, 0))

    q, kt, mt = pl.pallas_call(
        _proj_body,
        grid=(_N // _ROW_BLK,),
        in_specs=[blk, full, row, full, row, full, row, full, full],
        out_specs=[blk, blk, blk],
        out_shape=[jax.ShapeDtypeStruct((_N, _D), jnp.float32)] * 3,
    )(x, Wq, bq.reshape(1, _D), Wk, bk.reshape(1, _D), Wm, bm.reshape(1, _D),
      bwa, bwm)

    zeros = jnp.zeros((_N, _ACC_W), jnp.float32)
    acc = _edge_kernel()(kt, q, mt, src, dst, zeros)

    exp_mat = jnp.kron(jnp.eye(16, dtype=jnp.float32)[:, :_H],
                       jnp.ones((1, 16), jnp.float32))

    acc_blk = pl.BlockSpec((_NC, _ROW_BLK, _ACC_W), lambda i: (0, i, 0))
    out = pl.pallas_call(
        _out_body,
        grid=(_N // _ROW_BLK,),
        in_specs=[acc_blk, blk, full, row,
                  pl.BlockSpec(memory_space=pltpu.SMEM), row, row,
                  pl.BlockSpec((16, _D), lambda i: (0, 0))],
        out_specs=blk,
        out_shape=jax.ShapeDtypeStruct((_N, _D), jnp.float32),
    )(acc, x, Waggr, baggr.reshape(1, _D), skip_logit.reshape(1, 1),
      gamma.reshape(1, _D), beta.reshape(1, _D), exp_mat)
    return out


# speedup vs baseline: 23.5264x; 1.2017x over previous
import functools

import jax
import jax.numpy as jnp
from jax import lax
from jax.experimental import pallas as pl
from jax.experimental.pallas import tpu as pltpu
from jax.experimental.pallas import tpu_sc as plsc

_N = 10000
_E = 320000
_D = 128
_H = 8
_C = 16

_NC = 2
_NS = 16
_NW = _NC * _NS
_EPW = _E // _NW
_CHUNK = 40
_NSTEP = _EPW // _CHUNK
_ROWS_PT = _N // _NS
_ACC_W = _D + 16

_ROW_BLK = 1000


def _proj_body(x_ref, wq_ref, bq_ref, wk_ref, bk_ref, wm_ref, bm_ref,
               bwa_ref, bwm_ref, q_ref, kt_ref, mt_ref):
    xb = x_ref[...]
    f32 = jnp.float32
    q_ref[...] = jnp.dot(xb, wq_ref[...], preferred_element_type=f32) + bq_ref[...]
    kb = jnp.dot(xb, wk_ref[...], preferred_element_type=f32) + bk_ref[...]
    kt_ref[...] = jnp.dot(kb, bwa_ref[...], preferred_element_type=f32)
    mb = jnp.dot(xb, wm_ref[...], preferred_element_type=f32) + bm_ref[...]
    mt_ref[...] = jnp.dot(mb, bwm_ref[...], preferred_element_type=f32)


def _edge_body(kt_hbm, q_hbm, mt_hbm, src_hbm, dst_hbm, zeros_hbm, out_hbm,
               src_v, dst_v, ktb, qb, mtb, ob, acc, semi, semg):
    cid = lax.axis_index("c")
    sid = lax.axis_index("s")
    wid = cid * _NS + sid

    pltpu.sync_copy(zeros_hbm.at[pl.ds(sid * _ROWS_PT, _ROWS_PT)],
                    acc.at[pl.ds(sid * _ROWS_PT, _ROWS_PT)])
    plsc.subcore_barrier()

    base = wid * _EPW
    lane = lax.iota(jnp.int32, 16)

    def load_idx(g, p):
        off = base + g * _CHUNK
        pltpu.async_copy(src_hbm.at[pl.ds(off, _CHUNK)], src_v.at[p], semi)
        pltpu.async_copy(dst_hbm.at[pl.ds(off, _CHUNK)], dst_v.at[p], semi)

    def wait_idx(p):
        pltpu.make_async_copy(src_hbm.at[pl.ds(0, _CHUNK)], src_v.at[p], semi).wait()
        pltpu.make_async_copy(dst_hbm.at[pl.ds(0, _CHUNK)], dst_v.at[p], semi).wait()

    def issue_gathers(p):
        pltpu.async_copy(kt_hbm.at[src_v.at[p]], ktb.at[p], semg)
        pltpu.async_copy(q_hbm.at[dst_v.at[p]], qb.at[p], semg)
        pltpu.async_copy(mt_hbm.at[src_v.at[p]], mtb.at[p], semg)

    def wait_gathers(p):
        pltpu.make_async_copy(kt_hbm.at[src_v.at[p]], ktb.at[p], semg).wait()
        pltpu.make_async_copy(q_hbm.at[dst_v.at[p]], qb.at[p], semg).wait()
        pltpu.make_async_copy(mt_hbm.at[src_v.at[p]], mtb.at[p], semg).wait()

    load_idx(0, 0)
    wait_idx(0)
    issue_gathers(0)
    load_idx(1, 1)

    def step(g, carry):
        par = lax.rem(g, 2)
        nxt = 1 - par
        wait_gathers(par)

        @pl.when(g + 1 < _NSTEP)
        def _():
            wait_idx(nxt)
            issue_gathers(nxt)

        def edge(e, c2):
            dv = jnp.zeros((16,), jnp.float32)
            for h in range(_H):
                sl = pl.ds(h * 16, 16)
                ktv = ktb[par, e, sl]
                qv = qb[par, e, sl]
                s = jnp.sum(ktv * qv)
                ev = jnp.exp(jnp.full((16,), s, jnp.float32))
                ob[e, sl] = ev * mtb[par, e, sl]
                dv = jnp.where(lane == h, ev, dv)
            ob[e, pl.ds(_D, 16)] = dv
            return c2

        lax.fori_loop(0, _CHUNK, edge, 0)
        pltpu.sync_copy(ob, acc.at[dst_v.at[par]], add=True)

        @pl.when(g + 2 < _NSTEP)
        def _():
            load_idx(g + 2, par)

        return carry

    lax.fori_loop(0, _NSTEP, step, 0)
    plsc.subcore_barrier()
    pltpu.sync_copy(acc.at[pl.ds(sid * _ROWS_PT, _ROWS_PT)],
                    out_hbm.at[cid, pl.ds(sid * _ROWS_PT, _ROWS_PT)])


@functools.cache
def _edge_kernel():
    return functools.partial(
        pl.kernel,
        mesh=plsc.VectorSubcoreMesh(core_axis_name="c", subcore_axis_name="s"),
        compiler_params=pltpu.CompilerParams(use_tc_tiling_on_sc=False,
                                             needs_layout_passes=False),
        out_type=jax.ShapeDtypeStruct((_NC, _N, _ACC_W), jnp.float32),
        scratch_types=[
            pltpu.VMEM((2, _CHUNK), jnp.int32),
            pltpu.VMEM((2, _CHUNK), jnp.int32),
            pltpu.VMEM((2, _CHUNK, _D), jnp.float32),
            pltpu.VMEM((2, _CHUNK, _D), jnp.float32),
            pltpu.VMEM((2, _CHUNK, _D), jnp.float32),
            pltpu.VMEM((_CHUNK, _ACC_W), jnp.float32),
            pltpu.VMEM_SHARED((_N, _ACC_W), jnp.float32),
            pltpu.SemaphoreType.DMA,
            pltpu.SemaphoreType.DMA,
        ],
    )(_edge_body)


def _out_body(acc_ref, x_ref, waggr_ref, baggr_ref, slog_ref, gamma_ref,
              beta_ref, exp_ref, o_ref):
    a0 = acc_ref[0]
    a1 = acc_ref[1]
    num = a0[:, :_D] + a1[:, :_D]
    den = a0[:, _D:] + a1[:, _D:]
    den_exp = jnp.dot(den, exp_ref[...], preferred_element_type=jnp.float32)
    pooled = num / jnp.maximum(den_exp, 1e-30)
    act = jax.nn.gelu(pooled)
    aggr = jnp.dot(act, waggr_ref[...], preferred_element_type=jnp.float32)
    aggr = aggr + baggr_ref[...]
    gate = jax.nn.sigmoid(slog_ref[0, 0])
    out = gate * aggr + (1.0 - gate) * x_ref[...]
    mean = jnp.mean(out, axis=-1, keepdims=True)
    cent = out - mean
    var = jnp.mean(cent * cent, axis=-1, keepdims=True)
    out = cent * lax.rsqrt(var + 1e-3)
    o_ref[...] = gamma_ref[...] * out + beta_ref[...]


def kernel(x, edge_index, Wq, bq, Wk, bk, Wm, bm, Watt, Wmsg, prior, Waggr,
           baggr, skip_logit, gamma, beta):
    src = edge_index[0]
    dst = edge_index[1]

    eyeh = jnp.eye(_H, dtype=jnp.float32)
    watt_s = Watt * (prior / jnp.sqrt(jnp.float32(_C)))[:, None, None]
    bwa = jnp.einsum('hcd,hg->hcgd', watt_s, eyeh).reshape(_D, _D)
    bwm = jnp.einsum('hcd,hg->hcgd', Wmsg, eyeh).reshape(_D, _D)

    full = pl.BlockSpec((_D, _D), lambda i: (0, 0))
    row = pl.BlockSpec((1, _D), lambda i: (0, 0))
    blk = pl.BlockSpec((_ROW_BLK, _D), lambda i: (i, 0))

    q, kt, mt = pl.pallas_call(
        _proj_body,
        grid=(_N // _ROW_BLK,),
        in_specs=[blk, full, row, full, row, full, row, full, full],
        out_specs=[blk, blk, blk],
        out_shape=[jax.ShapeDtypeStruct((_N, _D), jnp.float32)] * 3,
    )(x, Wq, bq.reshape(1, _D), Wk, bk.reshape(1, _D), Wm, bm.reshape(1, _D),
      bwa, bwm)

    zeros = jnp.zeros((_N, _ACC_W), jnp.float32)
    acc = _edge_kernel()(kt, q, mt, src, dst, zeros)

    exp_mat = jnp.kron(jnp.eye(16, dtype=jnp.float32)[:, :_H],
                       jnp.ones((1, 16), jnp.float32))

    acc_blk = pl.BlockSpec((_NC, _ROW_BLK, _ACC_W), lambda i: (0, i, 0))
    out = pl.pallas_call(
        _out_body,
        grid=(_N // _ROW_BLK,),
        in_specs=[acc_blk, blk, full, row,
                  pl.BlockSpec(memory_space=pltpu.SMEM), row, row,
                  pl.BlockSpec((16, _D), lambda i: (0, 0))],
        out_specs=blk,
        out_shape=jax.ShapeDtypeStruct((_N, _D), jnp.float32),
    )(acc, x, Waggr, baggr.reshape(1, _D), skip_logit.reshape(1, 1),
      gamma.reshape(1, _D), beta.reshape(1, _D), exp_mat)
    return out


# speedup vs baseline: 97.7673x; 4.1557x over previous
import functools

import jax
import jax.numpy as jnp
from jax import lax
from jax.experimental import pallas as pl
from jax.experimental.pallas import tpu as pltpu
from jax.experimental.pallas import tpu_sc as plsc

_N = 10000
_E = 320000
_D = 128
_H = 8
_C = 16

_NC = 2
_NS = 16
_NW = _NC * _NS
_EPW = _E // _NW
_CHUNK = 40
_NSTEP = _EPW // _CHUNK
_ROWS_PT = _N // _NS
_ACC_W = _D + 16

_ROW_BLK = 1000


def _proj_body(x_ref, wq_ref, bq_ref, wk_ref, bk_ref, wm_ref, bm_ref,
               bwa_ref, bwm_ref, q_ref, kt_ref, mt_ref):
    xb = x_ref[...]
    f32 = jnp.float32
    qv = jnp.dot(xb, wq_ref[...], preferred_element_type=f32) + bq_ref[...]
    q_ref[...] = qv.astype(jnp.bfloat16)
    kb = jnp.dot(xb, wk_ref[...], preferred_element_type=f32) + bk_ref[...]
    kt_ref[...] = jnp.dot(kb, bwa_ref[...],
                          preferred_element_type=f32).astype(jnp.bfloat16)
    mb = jnp.dot(xb, wm_ref[...], preferred_element_type=f32) + bm_ref[...]
    mt_ref[...] = jnp.dot(mb, bwm_ref[...], preferred_element_type=f32)


def _edge_body(kt_hbm, q_hbm, mt_hbm, src_hbm, dst_hbm, zeros_hbm, out_hbm,
               src_v, dst_v, ktb, qb, mtb, ob, acc, semi, semg, sems):
    cid = lax.axis_index("c")
    sid = lax.axis_index("s")
    wid = cid * _NS + sid

    pltpu.sync_copy(zeros_hbm.at[pl.ds(sid * _ROWS_PT, _ROWS_PT)],
                    acc.at[pl.ds(sid * _ROWS_PT, _ROWS_PT)])
    plsc.subcore_barrier()

    base = wid * _EPW
    lane = lax.iota(jnp.int32, 16)
    bperms = [lane ^ s for s in (4, 2, 1)]
    pperm = [jnp.where(lane == 2 * p + 1, 8, 0) for p in range(_H // 2)]
    pmask = [lax.div(lane, 2) == p for p in range(_H // 2)]
    hsel = [lane * 0 + h for h in range(_H)]
    himask = jnp.int32(-65536)

    def load_idx(g, p4):
        off = base + g * _CHUNK
        pltpu.async_copy(src_hbm.at[pl.ds(off, _CHUNK)], src_v.at[lax.rem(p4, 2)], semi)
        pltpu.async_copy(dst_hbm.at[pl.ds(off, _CHUNK)], dst_v.at[p4], semi)

    def wait_idx(p4):
        pltpu.make_async_copy(src_hbm.at[pl.ds(0, _CHUNK)], src_v.at[lax.rem(p4, 2)], semi).wait()
        pltpu.make_async_copy(dst_hbm.at[pl.ds(0, _CHUNK)], dst_v.at[p4], semi).wait()

    def issue_gathers(p, p4):
        pltpu.async_copy(kt_hbm.at[src_v.at[p]], ktb.at[p], semg)
        pltpu.async_copy(q_hbm.at[dst_v.at[p4]], qb.at[p], semg)
        pltpu.async_copy(mt_hbm.at[src_v.at[p]], mtb.at[p], semg)

    def wait_gathers(p, p4):
        pltpu.make_async_copy(kt_hbm.at[src_v.at[p]], ktb.at[p], semg).wait()
        pltpu.make_async_copy(q_hbm.at[dst_v.at[p4]], qb.at[p], semg).wait()
        pltpu.make_async_copy(mt_hbm.at[src_v.at[p]], mtb.at[p], semg).wait()

    def wait_scatter(p, p4):
        pltpu.make_async_copy(ob.at[p], acc.at[dst_v.at[p4]], sems).wait()

    load_idx(0, 0)
    wait_idx(0)
    issue_gathers(0, 0)
    load_idx(1, 1)

    def step(g, carry):
        par = lax.rem(g, 2)
        nxt = 1 - par
        g4 = lax.rem(g, 4)
        wait_gathers(par, g4)

        @pl.when(g + 1 < _NSTEP)
        def _():
            wait_idx(lax.rem(g4 + 1, 4))
            issue_gathers(nxt, lax.rem(g4 + 1, 4))

        @pl.when(g >= 2)
        def _():
            wait_scatter(par, lax.rem(g4 + 2, 4))

        @plsc.parallel_loop(0, _CHUNK, unroll=2)
        def edge(e):
            svec = jnp.zeros((16,), jnp.float32)
            for p in range(_H // 2):
                sl = pl.ds(p * 16, 16)
                kv = ktb[par, e, sl]
                qv = qb[par, e, sl]
                klo = plsc.bitcast(kv << 16, jnp.float32)
                khi = plsc.bitcast(kv & himask, jnp.float32)
                qlo = plsc.bitcast(qv << 16, jnp.float32)
                qhi = plsc.bitcast(qv & himask, jnp.float32)
                t = klo * qlo + khi * qhi
                for pm in bperms:
                    t = t + t[pm]
                svec = jnp.where(pmask[p], t[pperm[p]], svec)
            es = jnp.exp(svec)
            ob[par, e, pl.ds(_D, 16)] = es
            for h in range(_H):
                sl = pl.ds(h * 16, 16)
                ob[par, e, sl] = es[hsel[h]] * mtb[par, e, sl]
        pltpu.async_copy(ob.at[par], acc.at[dst_v.at[g4]], sems, add=True)

        @pl.when(g + 2 < _NSTEP)
        def _():
            load_idx(g + 2, lax.rem(g4 + 2, 4))

        return carry

    lax.fori_loop(0, _NSTEP, step, 0)
    wait_scatter((_NSTEP - 2) % 2, (_NSTEP - 2) % 4)
    wait_scatter((_NSTEP - 1) % 2, (_NSTEP - 1) % 4)
    plsc.subcore_barrier()
    pltpu.sync_copy(acc.at[pl.ds(sid * _ROWS_PT, _ROWS_PT)],
                    out_hbm.at[cid, pl.ds(sid * _ROWS_PT, _ROWS_PT)])


@functools.cache
def _edge_kernel():
    return functools.partial(
        pl.kernel,
        mesh=plsc.VectorSubcoreMesh(core_axis_name="c", subcore_axis_name="s"),
        compiler_params=pltpu.CompilerParams(use_tc_tiling_on_sc=False,
                                             needs_layout_passes=False),
        out_type=jax.ShapeDtypeStruct((_NC, _N, _ACC_W), jnp.float32),
        scratch_types=[
            pltpu.VMEM((2, _CHUNK), jnp.int32),
            pltpu.VMEM((4, _CHUNK), jnp.int32),
            pltpu.VMEM((2, _CHUNK, _D // 2), jnp.int32),
            pltpu.VMEM((2, _CHUNK, _D // 2), jnp.int32),
            pltpu.VMEM((2, _CHUNK, _D), jnp.float32),
            pltpu.VMEM((2, _CHUNK, _ACC_W), jnp.float32),
            pltpu.VMEM_SHARED((_N, _ACC_W), jnp.float32),
            pltpu.SemaphoreType.DMA,
            pltpu.SemaphoreType.DMA,
            pltpu.SemaphoreType.DMA,
        ],
    )(_edge_body)


def _out_body(acc_ref, x_ref, waggr_ref, baggr_ref, slog_ref, gamma_ref,
              beta_ref, exp_ref, o_ref):
    a0 = acc_ref[0]
    a1 = acc_ref[1]
    num = a0[:, :_D] + a1[:, :_D]
    den = a0[:, _D:] + a1[:, _D:]
    den_exp = jnp.dot(den, exp_ref[...], preferred_element_type=jnp.float32)
    pooled = num / jnp.maximum(den_exp, 1e-30)
    act = jax.nn.gelu(pooled)
    aggr = jnp.dot(act, waggr_ref[...], preferred_element_type=jnp.float32)
    aggr = aggr + baggr_ref[...]
    gate = jax.nn.sigmoid(slog_ref[0, 0])
    out = gate * aggr + (1.0 - gate) * x_ref[...]
    mean = jnp.mean(out, axis=-1, keepdims=True)
    cent = out - mean
    var = jnp.mean(cent * cent, axis=-1, keepdims=True)
    out = cent * lax.rsqrt(var + 1e-3)
    o_ref[...] = gamma_ref[...] * out + beta_ref[...]


def kernel(x, edge_index, Wq, bq, Wk, bk, Wm, bm, Watt, Wmsg, prior, Waggr,
           baggr, skip_logit, gamma, beta):
    src = edge_index[0]
    dst = edge_index[1]

    eyeh = jnp.eye(_H, dtype=jnp.float32)
    watt_s = Watt * (prior / jnp.sqrt(jnp.float32(_C)))[:, None, None]
    bwa = jnp.einsum('hcd,hg->hcgd', watt_s, eyeh).reshape(_D, _D)
    bwm = jnp.einsum('hcd,hg->hcgd', Wmsg, eyeh).reshape(_D, _D)

    full = pl.BlockSpec((_D, _D), lambda i: (0, 0))
    row = pl.BlockSpec((1, _D), lambda i: (0, 0))
    blk = pl.BlockSpec((_ROW_BLK, _D), lambda i: (i, 0))

    q, kt, mt = pl.pallas_call(
        _proj_body,
        grid=(_N // _ROW_BLK,),
        in_specs=[blk, full, row, full, row, full, row, full, full],
        out_specs=[blk, blk, blk],
        out_shape=[jax.ShapeDtypeStruct((_N, _D), jnp.bfloat16),
                   jax.ShapeDtypeStruct((_N, _D), jnp.bfloat16),
                   jax.ShapeDtypeStruct((_N, _D), jnp.float32)],
    )(x, Wq, bq.reshape(1, _D), Wk, bk.reshape(1, _D), Wm, bm.reshape(1, _D),
      bwa, bwm)

    q32 = lax.bitcast_convert_type(q.reshape(_N, _D // 2, 2), jnp.int32)
    kt32 = lax.bitcast_convert_type(kt.reshape(_N, _D // 2, 2), jnp.int32)

    zeros = jnp.zeros((_N, _ACC_W), jnp.float32)
    acc = _edge_kernel()(kt32, q32, mt, src, dst, zeros)

    exp_mat = jnp.kron(jnp.eye(16, dtype=jnp.float32)[:, :_H],
                       jnp.ones((1, 16), jnp.float32))

    acc_blk = pl.BlockSpec((_NC, _ROW_BLK, _ACC_W), lambda i: (0, i, 0))
    out = pl.pallas_call(
        _out_body,
        grid=(_N // _ROW_BLK,),
        in_specs=[acc_blk, blk, full, row,
                  pl.BlockSpec(memory_space=pltpu.SMEM), row, row,
                  pl.BlockSpec((16, _D), lambda i: (0, 0))],
        out_specs=blk,
        out_shape=jax.ShapeDtypeStruct((_N, _D), jnp.float32),
    )(acc, x, Waggr, baggr.reshape(1, _D), skip_logit.reshape(1, 1),
      gamma.reshape(1, _D), beta.reshape(1, _D), exp_mat)
    return out
